# Initial kernel scaffold; baseline (speedup 1.0000x reference)
import jax
import jax.numpy as jnp
from jax.experimental import pallas as pl


def kernel(x, edge_index, W, b):
    raise NotImplementedError("write your pallas kernel here")


# speedup vs baseline: 4.1214x; 4.1214x over previous
import functools

import jax
import jax.numpy as jnp
from jax import lax
from jax.experimental import pallas as pl
from jax.experimental.pallas import tpu as pltpu
from jax.experimental.pallas import tpu_sc as plsc

NC = 2
NS = 16
C = 128
LANES = 16


def _matmul_body(x_ref, wd_ref, w2_ref, b_ref, p_ref, q_ref):
    xb = x_ref[...]
    h = xb.shape[1] // 2
    p = jnp.dot(xb, wd_ref[...], preferred_element_type=jnp.float32) + b_ref[...]
    q = jnp.dot(xb, w2_ref[...], preferred_element_type=jnp.float32)
    p_ref[:, 0, :] = p[:, :h]
    p_ref[:, 1, :] = p[:, h:]
    q_ref[:, 0, :] = q[:, :h]
    q_ref[:, 1, :] = q[:, h:]


def _combine_body(acc_ref, cnt_ref, o_ref):
    h = acc_ref.shape[2]
    r = 1.0 / jnp.maximum(cnt_ref[0, :, 0:1], 1.0)
    o_ref[:, :h] = acc_ref[0] * r
    o_ref[:, h:] = acc_ref[1] * r


def _make_sc_kernel(n_pad, d, chunks):
    rpt = n_pad // NS
    h = d // 2
    nj = h // LANES

    mesh = plsc.VectorSubcoreMesh(core_axis_name="c", subcore_axis_name="s")

    @functools.partial(
        pl.kernel,
        out_type=(
            jax.ShapeDtypeStruct((NC, n_pad, h), jnp.float32),
            jax.ShapeDtypeStruct((NC, n_pad, LANES), jnp.float32),
        ),
        mesh=mesh,
        scratch_types=[
            pltpu.VMEM((C,), jnp.int32),
            pltpu.VMEM((C,), jnp.int32),
            pltpu.VMEM((C,), jnp.int32),
            pltpu.VMEM((C,), jnp.int32),
            pltpu.VMEM((C, h), jnp.float32),
            pltpu.VMEM((C, h), jnp.float32),
            pltpu.VMEM((C, LANES), jnp.float32),
            pltpu.VMEM((C, LANES), jnp.float32),
            pltpu.VMEM_SHARED((n_pad, h), jnp.float32),
            pltpu.VMEM_SHARED((n_pad, LANES), jnp.float32),
            pltpu.SemaphoreType.DMA,
            pltpu.SemaphoreType.DMA,
        ],
        compiler_params=pltpu.CompilerParams(use_tc_tiling_on_sc=False),
    )
    def sc_kernel(p_hbm, q_hbm, row_hbm, col_hbm, acc_out, cnt_out,
                  ridx, cidx, gidx_r, gidx_c, bufp, bufq, ones_v, z16,
                  acc_sh, cnt_sh, sem_p, sem_q):
        cid = lax.axis_index("c")
        sid = lax.axis_index("s")

        zero = jnp.zeros((LANES,), jnp.float32)
        one = jnp.ones((LANES,), jnp.float32)

        def fill_row(r_, _):
            for j in range(nj):
                bufp[r_, pl.ds(j * LANES, LANES)] = zero
            ones_v[r_, pl.ds(0, LANES)] = one
            z16[r_, pl.ds(0, LANES)] = zero
            return 0

        lax.fori_loop(0, C, fill_row, 0)

        def zero_acc(k, _):
            r0 = sid * rpt + k * C
            pltpu.sync_copy(bufp, acc_sh.at[pl.ds(r0, C)])
            pltpu.sync_copy(z16, cnt_sh.at[pl.ds(r0, C)])
            return 0

        lax.fori_loop(0, rpt // C, zero_acc, 0)
        plsc.subcore_barrier()

        base = sid * chunks * C

        def do_chunk(k, _):
            off = base + k * C
            pltpu.sync_copy(row_hbm.at[pl.ds(off, C)], ridx)
            pltpu.sync_copy(col_hbm.at[pl.ds(off, C)], cidx)
            for j in range(C // LANES):
                sl = pl.ds(j * LANES, LANES)
                gidx_r[sl] = ridx[sl] * 2 + cid
                gidx_c[sl] = cidx[sl] * 2 + cid
            cp = pltpu.async_copy(p_hbm.at[gidx_r], bufp, sem_p)
            cq = pltpu.async_copy(q_hbm.at[gidx_c], bufq, sem_q)
            cp.wait()
            cq.wait()

            def relu_row(e, _):
                for j in range(nj):
                    sl = pl.ds(j * LANES, LANES)
                    bufp[e, sl] = jnp.maximum(bufp[e, sl] + bufq[e, sl], 0.0)
                return 0

            lax.fori_loop(0, C, relu_row, 0)
            pltpu.sync_copy(bufp, acc_sh.at[ridx], add=True)
            pltpu.sync_copy(ones_v, cnt_sh.at[ridx], add=True)
            return 0

        lax.fori_loop(0, chunks, do_chunk, 0)
        plsc.subcore_barrier()

        def write_out(k, _):
            r0 = sid * rpt + k * C
            pltpu.sync_copy(acc_sh.at[pl.ds(r0, C)], bufp)
            pltpu.sync_copy(bufp, acc_out.at[cid, pl.ds(r0, C)])
            pltpu.sync_copy(cnt_sh.at[pl.ds(r0, C)], z16)
            pltpu.sync_copy(z16, cnt_out.at[cid, pl.ds(r0, C)])
            return 0

        lax.fori_loop(0, rpt // C, write_out, 0)

    return sc_kernel


def kernel(x, edge_index, W, b):
    n, d = x.shape
    e = edge_index.shape[1]

    n_pad = ((n + 1 + NS * C - 1) // (NS * C)) * (NS * C)
    chunks = (e + NS * C - 1) // (NS * C)
    e_pad = chunks * NS * C

    wd = W[:d] - W[d:]
    w2 = W[d:]
    b2d = b[None, :]

    x_pad = jnp.pad(x, ((0, n_pad - n), (0, 0)))
    row_p = jnp.concatenate(
        [edge_index[0], jnp.full((e_pad - e,), n, dtype=jnp.int32)])
    col_p = jnp.concatenate(
        [edge_index[1], jnp.full((e_pad - e,), n, dtype=jnp.int32)])

    blk = 2048
    grid = n_pad // blk
    hw = d // 2
    p_tab, q_tab = pl.pallas_call(
        _matmul_body,
        grid=(grid,),
        in_specs=[
            pl.BlockSpec((blk, d), lambda i: (i, 0)),
            pl.BlockSpec((d, d), lambda i: (0, 0)),
            pl.BlockSpec((d, d), lambda i: (0, 0)),
            pl.BlockSpec((1, d), lambda i: (0, 0)),
        ],
        out_specs=[
            pl.BlockSpec((blk, NC, hw), lambda i: (i, 0, 0)),
            pl.BlockSpec((blk, NC, hw), lambda i: (i, 0, 0)),
        ],
        out_shape=[
            jax.ShapeDtypeStruct((n_pad, NC, hw), jnp.float32),
            jax.ShapeDtypeStruct((n_pad, NC, hw), jnp.float32),
        ],
    )(x_pad, wd, w2, b2d)
    p2 = p_tab.reshape(NC * n_pad, hw)
    q2 = q_tab.reshape(NC * n_pad, hw)

    sc_kernel = _make_sc_kernel(n_pad, d, chunks)
    acc, cnt = sc_kernel(p2, q2, row_p, col_p)

    out_pad = pl.pallas_call(
        _combine_body,
        grid=(grid,),
        in_specs=[
            pl.BlockSpec((NC, blk, hw), lambda i: (0, i, 0)),
            pl.BlockSpec((NC, blk, LANES), lambda i: (0, i, 0)),
        ],
        out_specs=pl.BlockSpec((blk, d), lambda i: (i, 0)),
        out_shape=jax.ShapeDtypeStruct((n_pad, d), jnp.float32),
    )(acc, cnt)

    return out_pad[:n]
